# baseline (speedup 1.0000x reference)
import jax
import jax.numpy as jnp
from jax import lax
from jax.experimental import pallas as pl
from jax.experimental.pallas import tpu as pltpu
from jax.experimental.pallas import tpu_sc as plsc

NUM_EMBEDDINGS = 1000000
EMBEDDING_DIM = 64
BATCH = 16384
HIST_LEN = 50

NC = 2
NS = 16
NW = NC * NS

L = 16
ILIST = 128
UBLK = 256
UNITS_PER_W = HIST_LEN * (BATCH // UBLK) // NW
GATHER_PAIRS = UNITS_PER_W // 2
UPH = BATCH // UBLK


def _iota16():
    return lax.iota(jnp.int32, L)


def _gather_body(tok_hbm, w_hbm, out_hbm, i0, i1, g0, g1, t0, t1,
                 sin0, sin1, sg0, sg1, so0, so1):
    wid = lax.axis_index("s") * NC + lax.axis_index("c")
    ub = wid * UNITS_PER_W

    def hu(uid):
        return lax.shift_right_logical(uid, 6), lax.bitwise_and(uid, UPH - 1)

    def in_copy(uid, i_ref, sem):
        h, u = hu(uid)
        return pltpu.make_async_copy(
            tok_hbm.at[h, pl.ds(u * UBLK, UBLK)], i_ref, sem)

    def g_copies(i_ref, g_ref, sem):
        return [
            pltpu.make_async_copy(
                w_hbm.at[i_ref.at[pl.ds(j * ILIST, ILIST)]],
                g_ref.at[pl.ds(j * ILIST, ILIST)], sem)
            for j in range(UBLK // ILIST)
        ]

    def out_copy(uid, t_ref, sem):
        h, u = hu(uid)
        return pltpu.make_async_copy(
            t_ref, out_hbm.at[h, :, pl.ds(u * UBLK, UBLK)], sem)

    def transpose(g_ref, t_ref):
        for m in range(UBLK // L):
            row_vec = _iota16() + (m * L)
            for c0 in range(0, EMBEDDING_DIM, 8):
                vals = [plsc.load_gather(
                            g_ref,
                            [row_vec, jnp.full((L,), c0 + c, jnp.int32)])
                        for c in range(8)]
                for i in range(8):
                    t_ref[c0 + i, pl.ds(m * L, L)] = vals[i]

    in_copy(ub, i0, sin0).start()
    in_copy(ub + 1, i1, sin1).start()
    in_copy(ub, i0, sin0).wait()
    for c in g_copies(i0, g0, sg0):
        c.start()
    in_copy(ub + 1, i1, sin1).wait()

    def itr(k, carry):
        u0 = ub + 2 * k
        for c in g_copies(i1, g1, sg1):
            c.start()
        for c in g_copies(i0, g0, sg0):
            c.wait()

        @pl.when(k > 0)
        def _():
            out_copy(u0 - 2, t0, so0).wait()
        transpose(g0, t0)
        out_copy(u0, t0, so0).start()

        @pl.when(k < GATHER_PAIRS - 1)
        def _():
            in_copy(u0 + 2, i0, sin0).start()
            in_copy(u0 + 2, i0, sin0).wait()
            for c in g_copies(i0, g0, sg0):
                c.start()

        for c in g_copies(i1, g1, sg1):
            c.wait()

        @pl.when(k > 0)
        def _():
            out_copy(u0 - 1, t1, so1).wait()
        transpose(g1, t1)
        out_copy(u0 + 1, t1, so1).start()

        @pl.when(k < GATHER_PAIRS - 1)
        def _():
            in_copy(u0 + 3, i1, sin1).start()
            in_copy(u0 + 3, i1, sin1).wait()
        return carry

    lax.fori_loop(0, GATHER_PAIRS, itr, 0)
    out_copy(ub + UNITS_PER_W - 2, t0, so0).wait()
    out_copy(ub + UNITS_PER_W - 1, t1, so1).wait()


def _gather(tok_t, weight):
    mesh = plsc.VectorSubcoreMesh(core_axis_name="c", subcore_axis_name="s")
    f = pl.kernel(
        _gather_body,
        out_type=jax.ShapeDtypeStruct((HIST_LEN, EMBEDDING_DIM, BATCH),
                                      jnp.float32),
        mesh=mesh,
        compiler_params=pltpu.CompilerParams(use_tc_tiling_on_sc=False,
                                             needs_layout_passes=False),
        scratch_types=[
            pltpu.VMEM((UBLK,), jnp.int32),
            pltpu.VMEM((UBLK,), jnp.int32),
            pltpu.VMEM((UBLK, EMBEDDING_DIM), jnp.float32),
            pltpu.VMEM((UBLK, EMBEDDING_DIM), jnp.float32),
            pltpu.VMEM((EMBEDDING_DIM, UBLK), jnp.float32),
            pltpu.VMEM((EMBEDDING_DIM, UBLK), jnp.float32),
            pltpu.SemaphoreType.DMA,
            pltpu.SemaphoreType.DMA,
            pltpu.SemaphoreType.DMA,
            pltpu.SemaphoreType.DMA,
            pltpu.SemaphoreType.DMA,
            pltpu.SemaphoreType.DMA,
        ],
    )
    return f(tok_t, weight)


@jax.jit
def _embed(token_ids, weight):
    tok_t = token_ids.astype(jnp.int32).T
    out3 = _gather(tok_t, weight)
    return out3.transpose(2, 0, 1)


def kernel(token_ids, weight):
    return _embed(token_ids, weight)


# speedup vs baseline: 1.0812x; 1.0812x over previous
import jax
import jax.numpy as jnp
from jax import lax
from jax.experimental import pallas as pl
from jax.experimental.pallas import tpu as pltpu
from jax.experimental.pallas import tpu_sc as plsc

NUM_EMBEDDINGS = 1000000
EMBEDDING_DIM = 64
BATCH = 16384
HIST_LEN = 50

NC = 2
NS = 16
NW = NC * NS

L = 16
ILIST = 128
UBLK = 256
UNITS_PER_W = HIST_LEN * (BATCH // UBLK) // NW
GATHER_PAIRS = UNITS_PER_W // 2
UPH = BATCH // UBLK


def _iota16():
    return lax.iota(jnp.int32, L)


def _gather_body(tok_hbm, w_hbm, out_hbm, i0, i1, g0, g1, gp0, gp1, t0, t1,
                 sin0, sin1, sg0, sg1, so0, so1):
    wid = lax.axis_index("s") * NC + lax.axis_index("c")
    ub = wid * UNITS_PER_W

    def hu(uid):
        return lax.shift_right_logical(uid, 6), lax.bitwise_and(uid, UPH - 1)

    def in_copy(uid, i_ref, sem):
        h, u = hu(uid)
        return pltpu.make_async_copy(
            tok_hbm.at[h, pl.ds(u * UBLK, UBLK)], i_ref, sem)

    def g_copies(i_ref, g_ref, sem):
        return [
            pltpu.make_async_copy(
                w_hbm.at[i_ref.at[pl.ds(j * ILIST, ILIST)]],
                g_ref.at[pl.ds(j * ILIST, ILIST)], sem)
            for j in range(UBLK // ILIST)
        ]

    def out_copy(uid, t_ref, sem):
        h, u = hu(uid)
        return pltpu.make_async_copy(
            t_ref, out_hbm.at[h, :, pl.ds(u * UBLK, UBLK)], sem)

    def transpose(g_ref, gp_ref, t_ref):
        for k0 in range(0, UBLK, 2):
            vs = [g_ref[k0 + (j // 4), pl.ds((j % 4) * L, L)]
                  for j in range(8)]
            for j in range(8):
                gp_ref[k0 + (j // 4), pl.ds((j % 4) * L, L)] = vs[j]
        for m in range(UBLK // L):
            row_vec = _iota16() + (m * L)
            for c0 in range(0, EMBEDDING_DIM, 8):
                vals = [plsc.load_gather(
                            gp_ref,
                            [row_vec, jnp.full((L,), c0 + c, jnp.int32)])
                        for c in range(8)]
                for i in range(8):
                    t_ref[c0 + i, pl.ds(m * L, L)] = vals[i]

    in_copy(ub, i0, sin0).start()
    in_copy(ub + 1, i1, sin1).start()
    in_copy(ub, i0, sin0).wait()
    for c in g_copies(i0, g0, sg0):
        c.start()
    in_copy(ub + 1, i1, sin1).wait()

    def itr(k, carry):
        u0 = ub + 2 * k
        for c in g_copies(i1, g1, sg1):
            c.start()
        for c in g_copies(i0, g0, sg0):
            c.wait()

        @pl.when(k > 0)
        def _():
            out_copy(u0 - 2, t0, so0).wait()
        transpose(g0, gp0, t0)
        out_copy(u0, t0, so0).start()

        @pl.when(k < GATHER_PAIRS - 1)
        def _():
            in_copy(u0 + 2, i0, sin0).start()
            in_copy(u0 + 2, i0, sin0).wait()
            for c in g_copies(i0, g0, sg0):
                c.start()

        for c in g_copies(i1, g1, sg1):
            c.wait()

        @pl.when(k > 0)
        def _():
            out_copy(u0 - 1, t1, so1).wait()
        transpose(g1, gp1, t1)
        out_copy(u0 + 1, t1, so1).start()

        @pl.when(k < GATHER_PAIRS - 1)
        def _():
            in_copy(u0 + 3, i1, sin1).start()
            in_copy(u0 + 3, i1, sin1).wait()
        return carry

    lax.fori_loop(0, GATHER_PAIRS, itr, 0)
    out_copy(ub + UNITS_PER_W - 2, t0, so0).wait()
    out_copy(ub + UNITS_PER_W - 1, t1, so1).wait()


def _gather(tok_t, weight):
    mesh = plsc.VectorSubcoreMesh(core_axis_name="c", subcore_axis_name="s")
    f = pl.kernel(
        _gather_body,
        out_type=jax.ShapeDtypeStruct((HIST_LEN, EMBEDDING_DIM, BATCH),
                                      jnp.float32),
        mesh=mesh,
        compiler_params=pltpu.CompilerParams(use_tc_tiling_on_sc=False,
                                             needs_layout_passes=False),
        scratch_types=[
            pltpu.VMEM((UBLK,), jnp.int32),
            pltpu.VMEM((UBLK,), jnp.int32),
            pltpu.VMEM((UBLK, EMBEDDING_DIM), jnp.float32),
            pltpu.VMEM((UBLK, EMBEDDING_DIM), jnp.float32),
            pltpu.VMEM((UBLK, EMBEDDING_DIM + 8), jnp.float32),
            pltpu.VMEM((UBLK, EMBEDDING_DIM + 8), jnp.float32),
            pltpu.VMEM((EMBEDDING_DIM, UBLK), jnp.float32),
            pltpu.VMEM((EMBEDDING_DIM, UBLK), jnp.float32),
            pltpu.SemaphoreType.DMA,
            pltpu.SemaphoreType.DMA,
            pltpu.SemaphoreType.DMA,
            pltpu.SemaphoreType.DMA,
            pltpu.SemaphoreType.DMA,
            pltpu.SemaphoreType.DMA,
        ],
    )
    return f(tok_t, weight)


@jax.jit
def _embed(token_ids, weight):
    tok_t = token_ids.astype(jnp.int32).T
    out3 = _gather(tok_t, weight)
    return out3.transpose(2, 0, 1)


def kernel(token_ids, weight):
    return _embed(token_ids, weight)


# speedup vs baseline: 1.2882x; 1.1915x over previous
import functools

import jax
import jax.numpy as jnp
from jax import lax
from jax.experimental import pallas as pl
from jax.experimental.pallas import tpu as pltpu
from jax.experimental.pallas import tpu_sc as plsc

NUM_EMBEDDINGS = 1000000
EMBEDDING_DIM = 64
BATCH = 16384
HIST_LEN = 50

NC = 2
NS = 16
NW = NC * NS

IDX_PER_STREAM = 128
TOTAL = BATCH * HIST_LEN
NROWS = TOTAL // IDX_PER_STREAM
ROWS_PER_W = NROWS // NW
CH = 4
STEPS = ROWS_PER_W // CH
CHUNK = CH * IDX_PER_STREAM


def _body(idx_hbm, w_hbm, out_hbm, idx_v, rows_v, sem):
    wid = lax.axis_index("s") * NC + lax.axis_index("c")
    row0 = wid * ROWS_PER_W

    def step(g, carry):
        r = row0 + g * CH
        pltpu.sync_copy(idx_hbm.at[pl.ds(r, CH)], idx_v)
        copies = []
        for j in range(CH):
            copies.append(pltpu.async_copy(
                w_hbm.at[idx_v.at[j]],
                rows_v.at[pl.ds(j * IDX_PER_STREAM, IDX_PER_STREAM)],
                sem))
        for c in copies:
            c.wait()
        pltpu.sync_copy(rows_v, out_hbm.at[pl.ds(r * IDX_PER_STREAM, CHUNK)])
        return carry

    lax.fori_loop(0, STEPS, step, 0)


@jax.jit
def _gather(idx2d, weight):
    mesh = plsc.VectorSubcoreMesh(core_axis_name="c", subcore_axis_name="s")
    f = pl.kernel(
        _body,
        out_type=jax.ShapeDtypeStruct((TOTAL, EMBEDDING_DIM), jnp.float32),
        mesh=mesh,
        compiler_params=pltpu.CompilerParams(use_tc_tiling_on_sc=False),
        scratch_types=[
            pltpu.VMEM((CH, IDX_PER_STREAM), jnp.int32),
            pltpu.VMEM((CHUNK, EMBEDDING_DIM), jnp.float32),
            pltpu.SemaphoreType.DMA,
        ],
    )
    return f(idx2d, weight)


def kernel(token_ids, weight):
    idx2d = token_ids.astype(jnp.int32).reshape(NROWS, IDX_PER_STREAM)
    out = _gather(idx2d, weight)
    return out.reshape(BATCH, HIST_LEN, EMBEDDING_DIM)
